# baseline (speedup 1.0000x reference)
import functools

import jax
import jax.numpy as jnp
from jax import lax
from jax.experimental import pallas as pl
from jax.experimental.pallas import tpu as pltpu

_C_BLK = 1024
_BASES = (0, 25, 50, 74)
_NBLKS = (25, 25, 24, 24)
_N_BLK = 25
_R = 32


def _stream_body(t_ref, x0_ref, x1_ref, x2_ref, x3_ref, loss_ref, m_ref,
                 s_ref, p_ref, *, c_blk, n_blk, v_total, n_rows):
    j = pl.program_id(0)
    x_refs = (x0_ref, x1_ref, x2_ref, x3_ref)
    neg_inf = jnp.float32(-jnp.inf)

    @pl.when(j == 0)
    def _():
        m_ref[...] = jnp.full(m_ref.shape, neg_inf, m_ref.dtype)
        s_ref[...] = jnp.zeros(s_ref.shape, s_ref.dtype)
        p_ref[...] = jnp.zeros(p_ref.shape, p_ref.dtype)

    blks = [_BASES[r] + jnp.minimum(j, _NBLKS[r] - 1) for r in range(4)]
    active = [True, True, j < _NBLKS[2], j < _NBLKS[3]]
    lane = lax.broadcasted_iota(jnp.int32, (1, 128), 1)
    n_chunks = c_blk // 128

    masks = [[None] * n_chunks for _ in range(4)]
    thr2 = jnp.where(active[2], jnp.int32(128), jnp.int32(0))
    for c in range(n_chunks):
        masks[2][c] = lane < thr2
    for c in range(n_chunks):
        thr = v_total - (blks[3] * c_blk + 128 * c)
        thr = jnp.where(active[3], thr, jnp.int32(0))
        masks[3][c] = lane < thr

    cbase = [jnp.where(active[r], blks[r] * c_blk, jnp.int32(-(2**28)))
             for r in range(4)]

    def tile(i, _):
        rows = pl.ds(i * _R, _R)
        t = t_ref[rows, :]
        m_old = m_ref[rows, :]
        m_new = m_old
        for r in range(4):
            for c in range(n_chunks):
                xc = x_refs[r][rows, pl.ds(128 * c, 128)]
                if masks[r][c] is not None:
                    xc = jnp.where(masks[r][c], xc, neg_inf)
                m_new = jnp.maximum(m_new, xc)
        s = s_ref[rows, :] * jnp.exp(m_old - m_new)
        p = p_ref[rows, :]
        for r in range(4):
            for c in range(n_chunks):
                xc = x_refs[r][rows, pl.ds(128 * c, 128)]
                xm = xc
                if masks[r][c] is not None:
                    xm = jnp.where(masks[r][c], xc, neg_inf)
                s = s + jnp.exp(xm - m_new)
                hit = (t - (cbase[r] + 128 * c)) == lane
                p = p + jnp.where(hit, xc, 0.0)
        m_ref[rows, :] = m_new
        s_ref[rows, :] = s
        p_ref[rows, :] = p
        return 0

    lax.fori_loop(0, n_rows // _R, tile, 0)

    @pl.when(j == n_blk - 1)
    def _():
        m128 = m_ref[...]
        big_m = jnp.max(m128, axis=1, keepdims=True)
        srow = jnp.sum(s_ref[...] * jnp.exp(m128 - big_m), axis=1,
                       keepdims=True)
        prow = jnp.sum(p_ref[...], axis=1, keepdims=True)
        loss_ref[...] = big_m + jnp.log(srow) - prow


def _topk_body(loss_ref, out_ref, *, k):
    loss = loss_ref[...]
    b = lax.bitcast_convert_type(loss, jnp.int32)
    key = b ^ (lax.shift_right_arithmetic(b, 31) & jnp.int32(0x7FFFFFFF))

    def cnt_ge(thresh):
        return jnp.sum((key >= thresh).astype(jnp.int32))

    base0 = jnp.where(cnt_ge(jnp.int32(0)) >= k, jnp.int32(0),
                      jnp.int32(-(2**31)))

    def body(i, base):
        cand = base | lax.shift_left(jnp.int32(1), 30 - i)
        return jnp.where(cnt_ge(cand) >= k, cand, base)

    big_t = lax.fori_loop(0, 31, body, base0)
    tb = big_t ^ (lax.shift_right_arithmetic(big_t, 31) & jnp.int32(0x7FFFFFFF))
    tval = lax.bitcast_convert_type(tb, jnp.float32)
    gt = loss > tval
    cnt_gt = jnp.sum(gt.astype(jnp.float32))
    sum_gt = jnp.sum(jnp.where(gt, loss, 0.0))
    res = (sum_gt + (jnp.float32(k) - cnt_gt) * tval) / jnp.float32(k)
    out_ref[...] = jnp.full((1, 1), res, jnp.float32)


@jax.jit
def kernel(inputs, targets):
    n, v = inputs.shape
    k = int(0.25 * n)
    t2 = targets.reshape(n, 1).astype(jnp.int32)

    def xspec(r):
        return pl.BlockSpec(
            (n, _C_BLK),
            lambda j, r=r: (0, _BASES[r] + jnp.minimum(j, _NBLKS[r] - 1)))

    loss = pl.pallas_call(
        functools.partial(_stream_body, c_blk=_C_BLK, n_blk=_N_BLK,
                          v_total=v, n_rows=n),
        grid=(_N_BLK,),
        in_specs=[pl.BlockSpec((n, 1), lambda j: (0, 0))] +
                 [xspec(r) for r in range(4)],
        out_specs=pl.BlockSpec((n, 1), lambda j: (0, 0)),
        out_shape=jax.ShapeDtypeStruct((n, 1), jnp.float32),
        scratch_shapes=[
            pltpu.VMEM((n, 128), jnp.float32),
            pltpu.VMEM((n, 128), jnp.float32),
            pltpu.VMEM((n, 128), jnp.float32),
        ],
        compiler_params=pltpu.CompilerParams(
            dimension_semantics=("arbitrary",)),
    )(t2, inputs, inputs, inputs, inputs)
    loss8 = loss.reshape(8, n // 8)
    out = pl.pallas_call(
        functools.partial(_topk_body, k=k),
        out_shape=jax.ShapeDtypeStruct((1, 1), jnp.float32),
    )(loss8)
    return out[0, 0]
